# baseline (speedup 1.0000x reference)
import functools

import jax
import jax.numpy as jnp
from jax import lax
from jax.experimental import pallas as pl
from jax.experimental.pallas import tpu as pltpu
from jax.experimental.pallas import tpu_sc as plsc


@functools.lru_cache(maxsize=None)
def _build(B, L, EMB):
    info = plsc.get_sparse_core_info()
    NC, NS = info.num_cores, info.num_subcores
    NW = NC * NS
    RPW = B // NW

    @functools.partial(
        pl.kernel,
        mesh=plsc.VectorSubcoreMesh(core_axis_name="c", subcore_axis_name="s"),
        compiler_params=pltpu.CompilerParams(use_tc_tiling_on_sc=False),
        out_type=jax.ShapeDtypeStruct((B, L, EMB), jnp.float32),
        scratch_types=[
            pltpu.VMEM((RPW, L), jnp.int32),
            pltpu.VMEM((2, L, EMB), jnp.float32),
            pltpu.VMEM((L, EMB), jnp.float32),
            pltpu.SemaphoreType.DMA,
            pltpu.SemaphoreType.DMA,
        ],
    )
    def k(x_hbm, emb_hbm, pos_hbm, out_hbm, idx_v, rows_v, pos_v, gsem, osem):
        wid = lax.axis_index("s") * NC + lax.axis_index("c")
        base = wid * RPW
        pltpu.sync_copy(pos_hbm.at[pl.ds(0, L)], pos_v)
        pltpu.sync_copy(x_hbm.at[pl.ds(base, RPW)], idx_v)

        def gather_start(b, buf):
            pltpu.async_copy(emb_hbm.at[idx_v.at[b]], rows_v.at[buf], gsem)

        def gather_wait():
            pltpu.make_async_copy(
                emb_hbm.at[idx_v.at[0]], rows_v.at[0], gsem).wait()

        def out_start(b, buf):
            pltpu.async_copy(rows_v.at[buf], out_hbm.at[base + b], osem)

        def out_wait():
            pltpu.make_async_copy(
                rows_v.at[0], out_hbm.at[base], osem).wait()

        def vadd(buf):
            def add_i(i, c):
                for j in range(EMB // 16):
                    sl = pl.ds(j * 16, 16)
                    rows_v[buf, i, sl] = rows_v[buf, i, sl] + pos_v[i, sl]
                return c
            lax.fori_loop(0, L, add_i, 0)

        gather_start(0, 0)
        gather_wait()
        gather_start(1, 1)
        vadd(0)
        out_start(0, 0)

        def step(b, c):
            cur = lax.rem(b, 2)
            nxt = 1 - cur
            gather_wait()
            out_wait()
            gather_start(b + 1, nxt)
            vadd(cur)
            out_start(b, cur)
            return c

        lax.fori_loop(1, RPW - 1, step, 0)

        gather_wait()
        out_wait()
        vadd((RPW - 1) % 2)
        out_start(RPW - 1, (RPW - 1) % 2)
        out_wait()

    return k


def kernel(x, emb_table, pos_table):
    B, L = x.shape
    EMB = emb_table.shape[1]
    k = _build(B, L, EMB)
    return k(x.astype(jnp.int32), emb_table, pos_table)


# speedup vs baseline: 1.3736x; 1.3736x over previous
import functools

import jax
import jax.numpy as jnp
from jax import lax
from jax.experimental import pallas as pl
from jax.experimental.pallas import tpu as pltpu
from jax.experimental.pallas import tpu_sc as plsc

CHUNK = 4


@functools.lru_cache(maxsize=None)
def _build(B, L, EMB):
    info = plsc.get_sparse_core_info()
    NC, NS = info.num_cores, info.num_subcores
    NW = NC * NS
    RPW = B // NW
    NCH = RPW // CHUNK

    @functools.partial(
        pl.kernel,
        mesh=plsc.VectorSubcoreMesh(core_axis_name="c", subcore_axis_name="s"),
        compiler_params=pltpu.CompilerParams(use_tc_tiling_on_sc=False),
        out_type=jax.ShapeDtypeStruct((B, L, EMB), jnp.float32),
        scratch_types=[
            pltpu.VMEM((CHUNK, L), jnp.int32),
            pltpu.VMEM((CHUNK, L), jnp.int32),
            pltpu.VMEM((CHUNK, L, EMB), jnp.float32),
            pltpu.VMEM((CHUNK, L, EMB), jnp.float32),
            pltpu.VMEM((L, EMB), jnp.float32),
            pltpu.SemaphoreType.DMA,
            pltpu.SemaphoreType.DMA,
            pltpu.SemaphoreType.DMA,
        ],
    )
    def k(x_hbm, emb_hbm, pos_hbm, out_hbm, ib0, ib1, rb0, rb1, pos_v,
          gsem, osem, isem):
        wid = lax.axis_index("s") * NC + lax.axis_index("c")
        base = wid * RPW
        ibufs = (ib0, ib1)
        rbufs = (rb0, rb1)

        def idx_start(c, ib):
            pltpu.async_copy(x_hbm.at[pl.ds(base + c * CHUNK, CHUNK)], ib, isem)

        def idx_wait():
            pltpu.make_async_copy(
                x_hbm.at[pl.ds(base, CHUNK)], ib0, isem).wait()

        def gather_start(ib, rb):
            for r in range(CHUNK):
                pltpu.async_copy(emb_hbm.at[ib.at[r]], rb.at[r], gsem)

        def gather_wait():
            for r in range(CHUNK):
                pltpu.make_async_copy(
                    emb_hbm.at[ib0.at[r]], rb0.at[r], gsem).wait()

        def out_start(c, rb):
            pltpu.async_copy(
                rb, out_hbm.at[pl.ds(base + c * CHUNK, CHUNK)], osem)

        def out_wait():
            pltpu.make_async_copy(
                rb0, out_hbm.at[pl.ds(base, CHUNK)], osem).wait()

        def vadd(rb):
            def add_i(i, carry):
                for j in range(EMB // 16):
                    sl = pl.ds(j * 16, 16)
                    p = pos_v[i, sl]
                    for r in range(CHUNK):
                        rb[r, i, sl] = rb[r, i, sl] + p
                return carry
            lax.fori_loop(0, L, add_i, 0)

        pltpu.sync_copy(pos_hbm.at[pl.ds(0, L)], pos_v)
        pltpu.sync_copy(x_hbm.at[pl.ds(base, CHUNK)], ib0)
        gather_start(ib0, rb0)
        idx_start(1, ib1)

        for c in range(NCH):
            A = c & 1
            if c + 1 < NCH:
                idx_wait()
                if c >= 1:
                    out_wait()
                gather_start(ibufs[1 - A], rbufs[1 - A])
            elif c >= 1:
                out_wait()
            gather_wait()
            if c + 2 < NCH:
                idx_start(c + 2, ibufs[A])
            vadd(rbufs[A])
            out_start(c, rbufs[A])
        out_wait()

    return k


def kernel(x, emb_table, pos_table):
    B, L = x.shape
    EMB = emb_table.shape[1]
    k = _build(B, L, EMB)
    return k(x.astype(jnp.int32), emb_table, pos_table)
